# baseline (speedup 1.0000x reference)
import functools

import jax
import jax.numpy as jnp
from jax.experimental import pallas as pl
from jax.experimental.pallas import tpu as pltpu

B = 128
N = 100000
BLK = 2048
NB = (N + BLK - 1) // BLK


def _reduce_body(act_ref, x_ref, norm_ref, mode_ref, pick_ref,
                 m_ref, s_ref, bv_ref, bi_ref, g_ref):
    i = pl.program_id(0)

    @pl.when(i == 0)
    def _init():
        m_ref[...] = jnp.full((B, 1), -jnp.inf, jnp.float32)
        s_ref[...] = jnp.zeros((B, 1), jnp.float32)
        bv_ref[...] = jnp.full((B, 1), -jnp.inf, jnp.float32)
        bi_ref[...] = jnp.zeros((B, 1), jnp.int32)
        g_ref[...] = jnp.zeros((B, 1), jnp.float32)

    x = x_ref[...]
    col0 = i * BLK
    cols = col0 + jax.lax.broadcasted_iota(jnp.int32, (B, BLK), 1)
    x = jnp.where(cols < N, x, -jnp.inf)

    bm = jnp.max(x, axis=1, keepdims=True)
    bi = jnp.argmax(x, axis=1)[:, None].astype(jnp.int32) + col0

    m_old = m_ref[...]
    m_new = jnp.maximum(m_old, bm)
    s_ref[...] = (s_ref[...] * jnp.exp(m_old - m_new)
                  + jnp.sum(jnp.exp(x - m_new), axis=1, keepdims=True))
    m_ref[...] = m_new

    better = bm > bv_ref[...]
    bv_ref[...] = jnp.where(better, bm, bv_ref[...])
    bi_ref[...] = jnp.where(better, bi, bi_ref[...])

    act = act_ref[...]
    hit = cols == act
    g_ref[...] += jnp.sum(jnp.where(hit, x, 0.0), axis=1, keepdims=True)

    @pl.when(i == NB - 1)
    def _fini():
        norm_ref[...] = m_ref[...] + jnp.log(s_ref[...])
        mode_ref[...] = bi_ref[...]
        pick_ref[...] = g_ref[...]


@jax.jit
def _tc_pass(logits, actions):
    norm, mode, pick = pl.pallas_call(
        _reduce_body,
        grid=(NB,),
        in_specs=[
            pl.BlockSpec((B, 1), lambda i: (0, 0)),
            pl.BlockSpec((B, BLK), lambda i: (0, i)),
        ],
        out_specs=[
            pl.BlockSpec((B, 1), lambda i: (0, 0)),
            pl.BlockSpec((B, 1), lambda i: (0, 0)),
            pl.BlockSpec((B, 1), lambda i: (0, 0)),
        ],
        out_shape=[
            jax.ShapeDtypeStruct((B, 1), jnp.float32),
            jax.ShapeDtypeStruct((B, 1), jnp.int32),
            jax.ShapeDtypeStruct((B, 1), jnp.float32),
        ],
        scratch_shapes=[
            pltpu.VMEM((B, 1), jnp.float32),
            pltpu.VMEM((B, 1), jnp.float32),
            pltpu.VMEM((B, 1), jnp.float32),
            pltpu.VMEM((B, 1), jnp.int32),
            pltpu.VMEM((B, 1), jnp.float32),
        ],
    )(actions, logits)
    return norm, mode, pick


def kernel(logits, actions):
    norm, mode, pick = _tc_pass(logits, actions)
    log_probs = pick - norm
    return (log_probs, mode)
